# baseline (speedup 1.0000x reference)
import functools

import jax
import jax.numpy as jnp
from jax import lax
from jax.experimental import pallas as pl
from jax.experimental.pallas import tpu as pltpu
from jax.experimental.pallas import tpu_sc as plsc

N = 10000
NP = 10240
E = 320000
D = 128
NC = 2
NS = 16
NW = NC * NS
CH = 128
NCHUNK = E // CH
CPW = -(-NCHUNK // NW) // 8 * 8 + (0 if NCHUNK % NW == 0 else 0)
CPW = 80
CPS = 16
EPAD = NW * CPW
ROWS_PT = NP // NS


def _sc_body(xp, xg, xs, epg, egp, esp, eps, zrows, zcnt, onesh,
             sums, cnts, acc, cacc, sall, dall, rows0, rows1, ones, g0, g1):
    c = lax.axis_index("c")
    s = lax.axis_index("s")
    wid = s * NC + c
    r0 = s * ROWS_PT
    pltpu.sync_copy(onesh, ones)

    for rel, (xsrc, ei) in enumerate(((xp, epg), (xg, egp), (xs, esp), (xp, eps))):
        pltpu.sync_copy(zrows, acc.at[pl.ds(r0, ROWS_PT)])
        pltpu.sync_copy(zcnt, cacc.at[pl.ds(r0, ROWS_PT)])
        plsc.subcore_barrier()

        for h in range(CPW // CPS):
            base = wid * CPW + h * CPS
            pltpu.sync_copy(ei.at[0, pl.ds(base, CPS)], sall)
            pltpu.sync_copy(ei.at[1, pl.ds(base, CPS)], dall)
            pltpu.async_copy(xsrc.at[sall.at[0]], rows0, g0)
            pltpu.async_copy(xsrc.at[sall.at[1]], rows1, g1)

            def halfstep(j, rows_p, gp):
                pltpu.make_async_copy(xsrc.at[pl.ds(0, CH)], rows_p, gp).wait()
                pltpu.sync_copy(rows_p, acc.at[dall.at[j]], add=True)
                pltpu.sync_copy(ones, cacc.at[dall.at[j]], add=True)
                nxt = jnp.minimum(j + 2, CPS - 1)
                pltpu.async_copy(xsrc.at[sall.at[nxt]], rows_p, gp)

            def body(i, _):
                halfstep(2 * i, rows0, g0)
                halfstep(2 * i + 1, rows1, g1)
                return 0

            lax.fori_loop(0, CPS // 2, body, 0)
            pltpu.make_async_copy(xsrc.at[pl.ds(0, CH)], rows0, g0).wait()
            pltpu.make_async_copy(xsrc.at[pl.ds(0, CH)], rows1, g1).wait()
        plsc.subcore_barrier()
        pltpu.sync_copy(acc.at[pl.ds(r0, ROWS_PT)],
                        sums.at[rel, c, pl.ds(r0, ROWS_PT)])
        pltpu.sync_copy(cacc.at[pl.ds(r0, ROWS_PT)],
                        cnts.at[rel, c, pl.ds(r0, ROWS_PT)])
        plsc.subcore_barrier()


_sc_aggregate = functools.partial(
    pl.kernel,
    out_type=[
        jax.ShapeDtypeStruct((4, NC, NP, D), jnp.float32),
        jax.ShapeDtypeStruct((4, NC, NP), jnp.float32),
    ],
    mesh=plsc.VectorSubcoreMesh(
        core_axis_name="c", subcore_axis_name="s",
        num_cores=NC, num_subcores=NS),
    scratch_types=[
        pltpu.VMEM_SHARED((NP, D), jnp.float32),
        pltpu.VMEM_SHARED((NP,), jnp.float32),
        pltpu.VMEM((CPS, CH), jnp.int32),
        pltpu.VMEM((CPS, CH), jnp.int32),
        pltpu.VMEM((CH, D), jnp.float32),
        pltpu.VMEM((CH, D), jnp.float32),
        pltpu.VMEM((CH,), jnp.float32),
        pltpu.SemaphoreType.DMA,
        pltpu.SemaphoreType.DMA,
    ],
)(_sc_body)


def _dot_t(a, b):
    return lax.dot_general(a, b, (((1,), (1,)), ((), ())),
                           preferred_element_type=jnp.float32)


def _tc_body(sums, cnts, xp, xg, xs,
             wl_pg, bl_pg, wr_pg, wl_gp, bl_gp, wr_gp,
             wl_ps, bl_ps, wr_ps, wl_sp, bl_sp, wr_sp,
             w_lin, b_lin, prelu_w,
             out_pfas, gw, sw):
    def mean(rel):
        sm = sums[rel, 0] + sums[rel, 1]
        cc = cnts[rel, 0] + cnts[rel, 1]
        return sm / jnp.maximum(cc, 1.0)

    o_gw = jnp.maximum(_dot_t(mean(0), wl_pg[...]) + bl_pg[...]
                       + _dot_t(xg[...], wr_pg[...]), 0.0)
    o_pf = (_dot_t(mean(1), wl_gp[...]) + bl_gp[...]
            + _dot_t(xp[...], wr_gp[...])
            + _dot_t(mean(2), wl_sp[...]) + bl_sp[...]
            + _dot_t(xp[...], wr_sp[...]))
    o_sw = jnp.maximum(_dot_t(mean(3), wl_ps[...]) + bl_ps[...]
                       + _dot_t(xs[...], wr_ps[...]), 0.0)
    out_pfas[...] = jnp.maximum(o_pf, 0.0)

    pw = prelu_w[0]
    bv = b_lin[0]
    g = jnp.sum(o_gw * w_lin[...], axis=1, keepdims=True) + bv
    t = jnp.sum(o_sw * w_lin[...], axis=1, keepdims=True) + bv
    gw[...] = jnp.maximum(g, 0.0) + pw * jnp.minimum(g, 0.0)
    sw[...] = jnp.maximum(t, 0.0) + pw * jnp.minimum(t, 0.0)


def _tc_dense(sums, cnts, xp, xg, xs, *weights):
    B = 2000
    grid = (N // B,)
    row = lambda i: (i, 0)
    w_spec = pl.BlockSpec((D, D), lambda i: (0, 0))
    b_spec = pl.BlockSpec((D,), lambda i: (0,))
    s_spec = pl.BlockSpec(memory_space=pltpu.SMEM)
    in_specs = [
        pl.BlockSpec((4, NC, B, D), lambda i: (0, 0, i, 0)),
        pl.BlockSpec((4, NC, B, 1), lambda i: (0, 0, i, 0)),
        pl.BlockSpec((B, D), row),
        pl.BlockSpec((B, D), row),
        pl.BlockSpec((B, D), row),
        w_spec, b_spec, w_spec,
        w_spec, b_spec, w_spec,
        w_spec, b_spec, w_spec,
        w_spec, b_spec, w_spec,
        pl.BlockSpec((1, D), lambda i: (0, 0)),
        s_spec, s_spec,
    ]
    out_specs = [
        pl.BlockSpec((B, D), row),
        pl.BlockSpec((B, 1), row),
        pl.BlockSpec((B, 1), row),
    ]
    out_shape = [
        jax.ShapeDtypeStruct((N, D), jnp.float32),
        jax.ShapeDtypeStruct((N, 1), jnp.float32),
        jax.ShapeDtypeStruct((N, 1), jnp.float32),
    ]
    return pl.pallas_call(
        _tc_body, grid=grid, in_specs=in_specs, out_specs=out_specs,
        out_shape=out_shape)(sums, cnts, xp, xg, xs, *weights)


def _prep_edges(ei):
    npad = EPAD * CH - E
    srcp = jnp.pad(ei[0:1], ((0, 0), (0, npad)))
    dump = N + (jnp.arange(npad, dtype=jnp.int32) % (NP - N))
    dstp = jnp.concatenate([ei[1:2], dump[None, :]], axis=1)
    return jnp.concatenate([srcp, dstp], axis=0).reshape(2, EPAD, CH)


def kernel(x_pfas_sites, x_gw_wells, x_sw_stations,
           edge_index_pg, edge_index_gp, edge_index_ps, edge_index_sp,
           Wl_pg, bl_pg, Wr_pg, Wl_gp, bl_gp, Wr_gp,
           Wl_ps, bl_ps, Wr_ps, Wl_sp, bl_sp, Wr_sp,
           W_lin, b_lin, prelu_w):
    zrows = jnp.zeros((ROWS_PT, D), jnp.float32)
    zcnt = jnp.zeros((ROWS_PT,), jnp.float32)
    onesh = jnp.ones((CH,), jnp.float32)
    sums, cnts = _sc_aggregate(
        x_pfas_sites, x_gw_wells, x_sw_stations,
        _prep_edges(edge_index_pg), _prep_edges(edge_index_gp),
        _prep_edges(edge_index_sp), _prep_edges(edge_index_ps),
        zrows, zcnt, onesh)
    cnts4 = cnts.reshape(4, NC, NP, 1)
    out_pfas, gw, sw = _tc_dense(
        sums, cnts4, x_pfas_sites, x_gw_wells, x_sw_stations,
        Wl_pg, bl_pg, Wr_pg, Wl_gp, bl_gp, Wr_gp,
        Wl_ps, bl_ps, Wr_ps, Wl_sp, bl_sp, Wr_sp,
        W_lin, b_lin, prelu_w)
    return out_pfas, gw, sw


# speedup vs baseline: 2.9954x; 2.9954x over previous
import functools

import jax
import jax.numpy as jnp
from jax import lax
from jax.experimental import pallas as pl
from jax.experimental.pallas import tpu as pltpu
from jax.experimental.pallas import tpu_sc as plsc

N = 10000
NP = 10240
E = 320000
D = 128
NC = 2
NS = 16
NW = NC * NS
CH = 128
NCHUNK = E // CH
ROWS_PT = NP // NS


def _sc_body(xp, xg, xs, epg, egp, esp, eps, zrows, zcnt, onesh,
             sums, cnts, acc, cacc, sidx0, didx0, sidx1, didx1,
             rows0, rows1, ones, g0, g1):
    c = lax.axis_index("c")
    s = lax.axis_index("s")
    wid = s * NC + c
    r0 = s * ROWS_PT
    pltpu.sync_copy(onesh, ones)

    base_n = NCHUNK // NW
    rem = NCHUNK - base_n * NW

    for rel, (xsrc, ei) in enumerate(((xp, epg), (xg, egp), (xs, esp), (xp, eps))):
        pltpu.sync_copy(zrows, acc.at[pl.ds(r0, ROWS_PT)])
        pltpu.sync_copy(zcnt, cacc.at[pl.ds(r0, ROWS_PT)])
        plsc.subcore_barrier()

        def load_idx(j, sidx_p, didx_p):
            pltpu.sync_copy(ei.at[0, pl.ds(j * CH, CH)], sidx_p)
            pltpu.sync_copy(ei.at[1, pl.ds(j * CH, CH)], didx_p)

        start = wid * base_n
        load_idx(start, sidx0, didx0)
        pltpu.async_copy(xsrc.at[sidx0], rows0, g0)
        load_idx(start + 1, sidx1, didx1)
        pltpu.async_copy(xsrc.at[sidx1], rows1, g1)

        def halfstep(j, sidx_p, didx_p, rows_p, gp):
            pltpu.make_async_copy(xsrc.at[pl.ds(0, CH)], rows_p, gp).wait()
            pltpu.sync_copy(rows_p, acc.at[didx_p], add=True)
            pltpu.sync_copy(ones, cacc.at[didx_p], add=True)
            nxt = jnp.minimum(j + 2, start + base_n - 1)
            load_idx(nxt, sidx_p, didx_p)
            pltpu.async_copy(xsrc.at[sidx_p], rows_p, gp)

        def body(i, _):
            halfstep(start + 2 * i, sidx0, didx0, rows0, g0)
            halfstep(start + 2 * i + 1, sidx1, didx1, rows1, g1)
            return 0

        lax.fori_loop(0, base_n // 2, body, 0)
        pltpu.make_async_copy(xsrc.at[pl.ds(0, CH)], rows0, g0).wait()
        pltpu.make_async_copy(xsrc.at[pl.ds(0, CH)], rows1, g1).wait()

        def tail_step():
            off = (NW * base_n + wid) * CH
            pltpu.sync_copy(ei.at[0, pl.ds(off, CH)], sidx0)
            pltpu.sync_copy(ei.at[1, pl.ds(off, CH)], didx0)
            pltpu.async_copy(xsrc.at[sidx0], rows0, g0).wait()
            pltpu.sync_copy(rows0, acc.at[didx0], add=True)
            pltpu.sync_copy(ones, cacc.at[didx0], add=True)

        pl.when(wid < rem)(tail_step)
        plsc.subcore_barrier()
        pltpu.sync_copy(acc.at[pl.ds(r0, ROWS_PT)],
                        sums.at[rel, c, pl.ds(r0, ROWS_PT)])
        pltpu.sync_copy(cacc.at[pl.ds(r0, ROWS_PT)],
                        cnts.at[rel, c, pl.ds(r0, ROWS_PT)])
        plsc.subcore_barrier()


_sc_aggregate = functools.partial(
    pl.kernel,
    out_type=[
        jax.ShapeDtypeStruct((4, NC, NP, D), jnp.float32),
        jax.ShapeDtypeStruct((4, NC, NP), jnp.float32),
    ],
    mesh=plsc.VectorSubcoreMesh(
        core_axis_name="c", subcore_axis_name="s",
        num_cores=NC, num_subcores=NS),
    scratch_types=[
        pltpu.VMEM_SHARED((NP, D), jnp.float32),
        pltpu.VMEM_SHARED((NP,), jnp.float32),
        pltpu.VMEM((CH,), jnp.int32),
        pltpu.VMEM((CH,), jnp.int32),
        pltpu.VMEM((CH,), jnp.int32),
        pltpu.VMEM((CH,), jnp.int32),
        pltpu.VMEM((CH, D), jnp.float32),
        pltpu.VMEM((CH, D), jnp.float32),
        pltpu.VMEM((CH,), jnp.float32),
        pltpu.SemaphoreType.DMA,
        pltpu.SemaphoreType.DMA,
    ],
)(_sc_body)


def _dot_t(a, b):
    return lax.dot_general(a, b, (((1,), (1,)), ((), ())),
                           preferred_element_type=jnp.float32)


def _tc_body(sums, cnts, xp, xg, xs,
             wl_pg, bl_pg, wr_pg, wl_gp, bl_gp, wr_gp,
             wl_ps, bl_ps, wr_ps, wl_sp, bl_sp, wr_sp,
             w_lin, b_lin, prelu_w,
             out_pfas, gw, sw):
    def mean(rel):
        sm = sums[rel, 0] + sums[rel, 1]
        cc = cnts[rel, 0] + cnts[rel, 1]
        return sm / jnp.maximum(cc, 1.0)

    o_gw = jnp.maximum(_dot_t(mean(0), wl_pg[...]) + bl_pg[...]
                       + _dot_t(xg[...], wr_pg[...]), 0.0)
    o_pf = (_dot_t(mean(1), wl_gp[...]) + bl_gp[...]
            + _dot_t(xp[...], wr_gp[...])
            + _dot_t(mean(2), wl_sp[...]) + bl_sp[...]
            + _dot_t(xp[...], wr_sp[...]))
    o_sw = jnp.maximum(_dot_t(mean(3), wl_ps[...]) + bl_ps[...]
                       + _dot_t(xs[...], wr_ps[...]), 0.0)
    out_pfas[...] = jnp.maximum(o_pf, 0.0)

    pw = prelu_w[0]
    bv = b_lin[0]
    g = jnp.sum(o_gw * w_lin[...], axis=1, keepdims=True) + bv
    t = jnp.sum(o_sw * w_lin[...], axis=1, keepdims=True) + bv
    gw[...] = jnp.maximum(g, 0.0) + pw * jnp.minimum(g, 0.0)
    sw[...] = jnp.maximum(t, 0.0) + pw * jnp.minimum(t, 0.0)


def _tc_dense(sums, cnts, xp, xg, xs, *weights):
    B = 2000
    grid = (N // B,)
    row = lambda i: (i, 0)
    w_spec = pl.BlockSpec((D, D), lambda i: (0, 0))
    b_spec = pl.BlockSpec((D,), lambda i: (0,))
    s_spec = pl.BlockSpec(memory_space=pltpu.SMEM)
    in_specs = [
        pl.BlockSpec((4, NC, B, D), lambda i: (0, 0, i, 0)),
        pl.BlockSpec((4, NC, B, 1), lambda i: (0, 0, i, 0)),
        pl.BlockSpec((B, D), row),
        pl.BlockSpec((B, D), row),
        pl.BlockSpec((B, D), row),
        w_spec, b_spec, w_spec,
        w_spec, b_spec, w_spec,
        w_spec, b_spec, w_spec,
        w_spec, b_spec, w_spec,
        pl.BlockSpec((1, D), lambda i: (0, 0)),
        s_spec, s_spec,
    ]
    out_specs = [
        pl.BlockSpec((B, D), row),
        pl.BlockSpec((B, 1), row),
        pl.BlockSpec((B, 1), row),
    ]
    out_shape = [
        jax.ShapeDtypeStruct((N, D), jnp.float32),
        jax.ShapeDtypeStruct((N, 1), jnp.float32),
        jax.ShapeDtypeStruct((N, 1), jnp.float32),
    ]
    return pl.pallas_call(
        _tc_body, grid=grid, in_specs=in_specs, out_specs=out_specs,
        out_shape=out_shape)(sums, cnts, xp, xg, xs, *weights)


def kernel(x_pfas_sites, x_gw_wells, x_sw_stations,
           edge_index_pg, edge_index_gp, edge_index_ps, edge_index_sp,
           Wl_pg, bl_pg, Wr_pg, Wl_gp, bl_gp, Wr_gp,
           Wl_ps, bl_ps, Wr_ps, Wl_sp, bl_sp, Wr_sp,
           W_lin, b_lin, prelu_w):
    zrows = jnp.zeros((ROWS_PT, D), jnp.float32)
    zcnt = jnp.zeros((ROWS_PT,), jnp.float32)
    onesh = jnp.ones((CH,), jnp.float32)
    sums, cnts = _sc_aggregate(
        x_pfas_sites, x_gw_wells, x_sw_stations,
        edge_index_pg, edge_index_gp, edge_index_sp, edge_index_ps,
        zrows, zcnt, onesh)
    cnts4 = cnts.reshape(4, NC, NP, 1)
    out_pfas, gw, sw = _tc_dense(
        sums, cnts4, x_pfas_sites, x_gw_wells, x_sw_stations,
        Wl_pg, bl_pg, Wr_pg, Wl_gp, bl_gp, Wr_gp,
        Wl_ps, bl_ps, Wr_ps, Wl_sp, bl_sp, Wr_sp,
        W_lin, b_lin, prelu_w)
    return out_pfas, gw, sw


# speedup vs baseline: 3.6125x; 1.2060x over previous
import functools

import jax
import jax.numpy as jnp
from jax import lax
from jax.experimental import pallas as pl
from jax.experimental.pallas import tpu as pltpu
from jax.experimental.pallas import tpu_sc as plsc

N = 10000
NP = 10240
E = 320000
D = 128
NC = 2
NS = 16
NW = NC * NS
CH = 128
NCHUNK = E // CH
ROWS_PT = NP // NS


def _sc_body(xp, xg, xs, epg, egp, esp, eps, zrows, zcnt, onesh,
             sums, cnts, acc, cacc, sidx0, didx0, sidx1, didx1,
             rows0, rows1, ones, g0, g1, s0, s1):
    c = lax.axis_index("c")
    s = lax.axis_index("s")
    wid = s * NC + c
    r0 = s * ROWS_PT
    pltpu.sync_copy(onesh, ones)

    base_n = NCHUNK // NW
    rem = NCHUNK - base_n * NW

    for rel, (xsrc, ei) in enumerate(((xp, epg), (xg, egp), (xs, esp), (xp, eps))):
        pltpu.sync_copy(zrows, acc.at[pl.ds(r0, ROWS_PT)])
        pltpu.sync_copy(zcnt, cacc.at[pl.ds(r0, ROWS_PT)])
        plsc.subcore_barrier()

        def load_idx(j, sidx_p, didx_p):
            pltpu.sync_copy(ei.at[0, pl.ds(j * CH, CH)], sidx_p)
            pltpu.sync_copy(ei.at[1, pl.ds(j * CH, CH)], didx_p)

        start = wid * base_n
        load_idx(start, sidx0, didx0)
        pltpu.async_copy(xsrc.at[sidx0], rows0, g0)
        load_idx(start + 1, sidx1, didx1)
        pltpu.async_copy(xsrc.at[sidx1], rows1, g1)

        def halfstep(j, sidx_p, didx_p, rows_p, gp, sp):
            pltpu.make_async_copy(xsrc.at[pl.ds(0, CH)], rows_p, gp).wait()
            pltpu.async_copy(rows_p, acc.at[didx_p], sp, add=True)
            pltpu.async_copy(ones, cacc.at[didx_p], sp, add=True)
            nxt = jnp.minimum(j + 2, start + base_n - 1)
            pltpu.sync_copy(ei.at[0, pl.ds(nxt * CH, CH)], sidx_p)
            pltpu.make_async_copy(rows_p, acc.at[pl.ds(0, CH)], sp).wait()
            pltpu.make_async_copy(ones, cacc.at[pl.ds(0, CH)], sp).wait()
            pltpu.sync_copy(ei.at[1, pl.ds(nxt * CH, CH)], didx_p)
            pltpu.async_copy(xsrc.at[sidx_p], rows_p, gp)

        def body(i, _):
            halfstep(start + 2 * i, sidx0, didx0, rows0, g0, s0)
            halfstep(start + 2 * i + 1, sidx1, didx1, rows1, g1, s1)
            return 0

        lax.fori_loop(0, base_n // 2, body, 0)
        pltpu.make_async_copy(xsrc.at[pl.ds(0, CH)], rows0, g0).wait()
        pltpu.make_async_copy(xsrc.at[pl.ds(0, CH)], rows1, g1).wait()

        def tail_step():
            off = (NW * base_n + wid) * CH
            pltpu.sync_copy(ei.at[0, pl.ds(off, CH)], sidx0)
            pltpu.sync_copy(ei.at[1, pl.ds(off, CH)], didx0)
            pltpu.async_copy(xsrc.at[sidx0], rows0, g0).wait()
            pltpu.sync_copy(rows0, acc.at[didx0], add=True)
            pltpu.sync_copy(ones, cacc.at[didx0], add=True)

        pl.when(wid < rem)(tail_step)
        plsc.subcore_barrier()
        pltpu.sync_copy(acc.at[pl.ds(r0, ROWS_PT)],
                        sums.at[rel, c, pl.ds(r0, ROWS_PT)])
        pltpu.sync_copy(cacc.at[pl.ds(r0, ROWS_PT)],
                        cnts.at[rel, c, pl.ds(r0, ROWS_PT)])
        plsc.subcore_barrier()


_sc_aggregate = functools.partial(
    pl.kernel,
    out_type=[
        jax.ShapeDtypeStruct((4, NC, NP, D), jnp.float32),
        jax.ShapeDtypeStruct((4, NC, NP), jnp.float32),
    ],
    mesh=plsc.VectorSubcoreMesh(
        core_axis_name="c", subcore_axis_name="s",
        num_cores=NC, num_subcores=NS),
    scratch_types=[
        pltpu.VMEM_SHARED((NP, D), jnp.float32),
        pltpu.VMEM_SHARED((NP,), jnp.float32),
        pltpu.VMEM((CH,), jnp.int32),
        pltpu.VMEM((CH,), jnp.int32),
        pltpu.VMEM((CH,), jnp.int32),
        pltpu.VMEM((CH,), jnp.int32),
        pltpu.VMEM((CH, D), jnp.float32),
        pltpu.VMEM((CH, D), jnp.float32),
        pltpu.VMEM((CH,), jnp.float32),
        pltpu.SemaphoreType.DMA,
        pltpu.SemaphoreType.DMA,
        pltpu.SemaphoreType.DMA,
        pltpu.SemaphoreType.DMA,
    ],
)(_sc_body)


def _dot_t(a, b):
    return lax.dot_general(a, b, (((1,), (1,)), ((), ())),
                           preferred_element_type=jnp.float32)


def _tc_body(sums, cnts, xp, xg, xs,
             wl_pg, bl_pg, wr_pg, wl_gp, bl_gp, wr_gp,
             wl_ps, bl_ps, wr_ps, wl_sp, bl_sp, wr_sp,
             w_lin, b_lin, prelu_w,
             out_pfas, gw, sw):
    def mean(rel):
        sm = sums[rel, 0] + sums[rel, 1]
        cc = cnts[rel, 0] + cnts[rel, 1]
        return sm / jnp.maximum(cc, 1.0)

    o_gw = jnp.maximum(_dot_t(mean(0), wl_pg[...]) + bl_pg[...]
                       + _dot_t(xg[...], wr_pg[...]), 0.0)
    o_pf = (_dot_t(mean(1), wl_gp[...]) + bl_gp[...]
            + _dot_t(xp[...], wr_gp[...])
            + _dot_t(mean(2), wl_sp[...]) + bl_sp[...]
            + _dot_t(xp[...], wr_sp[...]))
    o_sw = jnp.maximum(_dot_t(mean(3), wl_ps[...]) + bl_ps[...]
                       + _dot_t(xs[...], wr_ps[...]), 0.0)
    out_pfas[...] = jnp.maximum(o_pf, 0.0)

    pw = prelu_w[0]
    bv = b_lin[0]
    g = jnp.sum(o_gw * w_lin[...], axis=1, keepdims=True) + bv
    t = jnp.sum(o_sw * w_lin[...], axis=1, keepdims=True) + bv
    gw[...] = jnp.maximum(g, 0.0) + pw * jnp.minimum(g, 0.0)
    sw[...] = jnp.maximum(t, 0.0) + pw * jnp.minimum(t, 0.0)


def _tc_dense(sums, cnts, xp, xg, xs, *weights):
    B = 2000
    grid = (N // B,)
    row = lambda i: (i, 0)
    w_spec = pl.BlockSpec((D, D), lambda i: (0, 0))
    b_spec = pl.BlockSpec((D,), lambda i: (0,))
    s_spec = pl.BlockSpec(memory_space=pltpu.SMEM)
    in_specs = [
        pl.BlockSpec((4, NC, B, D), lambda i: (0, 0, i, 0)),
        pl.BlockSpec((4, NC, B, 1), lambda i: (0, 0, i, 0)),
        pl.BlockSpec((B, D), row),
        pl.BlockSpec((B, D), row),
        pl.BlockSpec((B, D), row),
        w_spec, b_spec, w_spec,
        w_spec, b_spec, w_spec,
        w_spec, b_spec, w_spec,
        w_spec, b_spec, w_spec,
        pl.BlockSpec((1, D), lambda i: (0, 0)),
        s_spec, s_spec,
    ]
    out_specs = [
        pl.BlockSpec((B, D), row),
        pl.BlockSpec((B, 1), row),
        pl.BlockSpec((B, 1), row),
    ]
    out_shape = [
        jax.ShapeDtypeStruct((N, D), jnp.float32),
        jax.ShapeDtypeStruct((N, 1), jnp.float32),
        jax.ShapeDtypeStruct((N, 1), jnp.float32),
    ]
    return pl.pallas_call(
        _tc_body, grid=grid, in_specs=in_specs, out_specs=out_specs,
        out_shape=out_shape)(sums, cnts, xp, xg, xs, *weights)


def kernel(x_pfas_sites, x_gw_wells, x_sw_stations,
           edge_index_pg, edge_index_gp, edge_index_ps, edge_index_sp,
           Wl_pg, bl_pg, Wr_pg, Wl_gp, bl_gp, Wr_gp,
           Wl_ps, bl_ps, Wr_ps, Wl_sp, bl_sp, Wr_sp,
           W_lin, b_lin, prelu_w):
    zrows = jnp.zeros((ROWS_PT, D), jnp.float32)
    zcnt = jnp.zeros((ROWS_PT,), jnp.float32)
    onesh = jnp.ones((CH,), jnp.float32)
    sums, cnts = _sc_aggregate(
        x_pfas_sites, x_gw_wells, x_sw_stations,
        edge_index_pg, edge_index_gp, edge_index_sp, edge_index_ps,
        zrows, zcnt, onesh)
    cnts4 = cnts.reshape(4, NC, NP, 1)
    out_pfas, gw, sw = _tc_dense(
        sums, cnts4, x_pfas_sites, x_gw_wells, x_sw_stations,
        Wl_pg, bl_pg, Wr_pg, Wl_gp, bl_gp, Wr_gp,
        Wl_ps, bl_ps, Wr_ps, Wl_sp, bl_sp, Wr_sp,
        W_lin, b_lin, prelu_w)
    return out_pfas, gw, sw


# speedup vs baseline: 3.6970x; 1.0234x over previous
import functools

import jax
import jax.numpy as jnp
from jax import lax
from jax.experimental import pallas as pl
from jax.experimental.pallas import tpu as pltpu
from jax.experimental.pallas import tpu_sc as plsc

N = 10000
NP = 10240
E = 320000
D = 128
NC = 2
NS = 16
NW = NC * NS
CH = 128
NCHUNK = E // CH
ROWS_PT = NP // NS


def _sc_body(xp, xg, xs, epg, egp, esp, eps, zrows, zcnt, onesh,
             sums, cnts, acc, cacc, sidx0, didx0, sidx1, didx1,
             rows0, rows1, ones, g0, g1, s0, s1):
    c = lax.axis_index("c")
    s = lax.axis_index("s")
    wid = s * NC + c
    r0 = s * ROWS_PT
    pltpu.sync_copy(onesh, ones)

    base_n = NCHUNK // NW
    rem = NCHUNK - base_n * NW

    for rel, (xsrc, ei) in enumerate(((xp, epg), (xg, egp), (xs, esp), (xp, eps))):
        pltpu.sync_copy(zrows, acc.at[pl.ds(r0, ROWS_PT)])
        pltpu.sync_copy(zcnt, cacc.at[pl.ds(r0, ROWS_PT)])
        plsc.subcore_barrier()

        start = wid * base_n
        pltpu.sync_copy(ei.at[0, pl.ds(start * CH, CH)], sidx0)
        pltpu.sync_copy(ei.at[1, pl.ds(start * CH, CH)], didx0.at[0])
        pltpu.async_copy(xsrc.at[sidx0], rows0, g0)
        pltpu.sync_copy(ei.at[0, pl.ds((start + 1) * CH, CH)], sidx1)
        pltpu.sync_copy(ei.at[1, pl.ds((start + 1) * CH, CH)], didx1.at[0])
        pltpu.async_copy(xsrc.at[sidx1], rows1, g1)

        def halfstep(j, cur, sidx_p, didx_p, rows_p, gp, sp):
            pltpu.make_async_copy(xsrc.at[pl.ds(0, CH)], rows_p, gp).wait()
            pltpu.async_copy(rows_p, acc.at[didx_p.at[cur]], sp, add=True)
            pltpu.async_copy(ones, cacc.at[didx_p.at[cur]], sp, add=True)
            nxt = jnp.minimum(j + 2, start + base_n - 1)
            pltpu.sync_copy(ei.at[0, pl.ds(nxt * CH, CH)], sidx_p)
            pltpu.sync_copy(ei.at[1, pl.ds(nxt * CH, CH)], didx_p.at[1 - cur])
            pltpu.make_async_copy(rows_p, acc.at[pl.ds(0, CH)], sp).wait()
            pltpu.make_async_copy(ones, cacc.at[pl.ds(0, CH)], sp).wait()
            pltpu.async_copy(xsrc.at[sidx_p], rows_p, gp)

        def body(i, _):
            cur = i & 1
            halfstep(start + 2 * i, cur, sidx0, didx0, rows0, g0, s0)
            halfstep(start + 2 * i + 1, cur, sidx1, didx1, rows1, g1, s1)
            return 0

        lax.fori_loop(0, base_n // 2, body, 0)
        pltpu.make_async_copy(xsrc.at[pl.ds(0, CH)], rows0, g0).wait()
        pltpu.make_async_copy(xsrc.at[pl.ds(0, CH)], rows1, g1).wait()

        def tail_step():
            off = (NW * base_n + wid) * CH
            pltpu.sync_copy(ei.at[0, pl.ds(off, CH)], sidx0)
            pltpu.sync_copy(ei.at[1, pl.ds(off, CH)], didx0.at[0])
            pltpu.async_copy(xsrc.at[sidx0], rows0, g0).wait()
            pltpu.sync_copy(rows0, acc.at[didx0.at[0]], add=True)
            pltpu.sync_copy(ones, cacc.at[didx0.at[0]], add=True)

        pl.when(wid < rem)(tail_step)
        plsc.subcore_barrier()
        pltpu.sync_copy(acc.at[pl.ds(r0, ROWS_PT)],
                        sums.at[rel, c, pl.ds(r0, ROWS_PT)])
        pltpu.sync_copy(cacc.at[pl.ds(r0, ROWS_PT)],
                        cnts.at[rel, c, pl.ds(r0, ROWS_PT)])
        plsc.subcore_barrier()


_sc_aggregate = functools.partial(
    pl.kernel,
    out_type=[
        jax.ShapeDtypeStruct((4, NC, NP, D), jnp.float32),
        jax.ShapeDtypeStruct((4, NC, NP), jnp.float32),
    ],
    mesh=plsc.VectorSubcoreMesh(
        core_axis_name="c", subcore_axis_name="s",
        num_cores=NC, num_subcores=NS),
    scratch_types=[
        pltpu.VMEM_SHARED((NP, D), jnp.float32),
        pltpu.VMEM_SHARED((NP,), jnp.float32),
        pltpu.VMEM((CH,), jnp.int32),
        pltpu.VMEM((2, CH), jnp.int32),
        pltpu.VMEM((CH,), jnp.int32),
        pltpu.VMEM((2, CH), jnp.int32),
        pltpu.VMEM((CH, D), jnp.float32),
        pltpu.VMEM((CH, D), jnp.float32),
        pltpu.VMEM((CH,), jnp.float32),
        pltpu.SemaphoreType.DMA,
        pltpu.SemaphoreType.DMA,
        pltpu.SemaphoreType.DMA,
        pltpu.SemaphoreType.DMA,
    ],
)(_sc_body)


def _dot_t(a, b):
    return lax.dot_general(a, b, (((1,), (1,)), ((), ())),
                           preferred_element_type=jnp.float32)


def _tc_body(sums, cnts, xp, xg, xs,
             wl_pg, bl_pg, wr_pg, wl_gp, bl_gp, wr_gp,
             wl_ps, bl_ps, wr_ps, wl_sp, bl_sp, wr_sp,
             w_lin, b_lin, prelu_w,
             out_pfas, gw, sw):
    def mean(rel):
        sm = sums[rel, 0] + sums[rel, 1]
        cc = cnts[rel, 0] + cnts[rel, 1]
        return sm / jnp.maximum(cc, 1.0)

    o_gw = jnp.maximum(_dot_t(mean(0), wl_pg[...]) + bl_pg[...]
                       + _dot_t(xg[...], wr_pg[...]), 0.0)
    o_pf = (_dot_t(mean(1), wl_gp[...]) + bl_gp[...]
            + _dot_t(xp[...], wr_gp[...])
            + _dot_t(mean(2), wl_sp[...]) + bl_sp[...]
            + _dot_t(xp[...], wr_sp[...]))
    o_sw = jnp.maximum(_dot_t(mean(3), wl_ps[...]) + bl_ps[...]
                       + _dot_t(xs[...], wr_ps[...]), 0.0)
    out_pfas[...] = jnp.maximum(o_pf, 0.0)

    pw = prelu_w[0]
    bv = b_lin[0]
    g = jnp.sum(o_gw * w_lin[...], axis=1, keepdims=True) + bv
    t = jnp.sum(o_sw * w_lin[...], axis=1, keepdims=True) + bv
    gw[...] = jnp.maximum(g, 0.0) + pw * jnp.minimum(g, 0.0)
    sw[...] = jnp.maximum(t, 0.0) + pw * jnp.minimum(t, 0.0)


def _tc_dense(sums, cnts, xp, xg, xs, *weights):
    B = 2000
    grid = (N // B,)
    row = lambda i: (i, 0)
    w_spec = pl.BlockSpec((D, D), lambda i: (0, 0))
    b_spec = pl.BlockSpec((D,), lambda i: (0,))
    s_spec = pl.BlockSpec(memory_space=pltpu.SMEM)
    in_specs = [
        pl.BlockSpec((4, NC, B, D), lambda i: (0, 0, i, 0)),
        pl.BlockSpec((4, NC, B, 1), lambda i: (0, 0, i, 0)),
        pl.BlockSpec((B, D), row),
        pl.BlockSpec((B, D), row),
        pl.BlockSpec((B, D), row),
        w_spec, b_spec, w_spec,
        w_spec, b_spec, w_spec,
        w_spec, b_spec, w_spec,
        w_spec, b_spec, w_spec,
        pl.BlockSpec((1, D), lambda i: (0, 0)),
        s_spec, s_spec,
    ]
    out_specs = [
        pl.BlockSpec((B, D), row),
        pl.BlockSpec((B, 1), row),
        pl.BlockSpec((B, 1), row),
    ]
    out_shape = [
        jax.ShapeDtypeStruct((N, D), jnp.float32),
        jax.ShapeDtypeStruct((N, 1), jnp.float32),
        jax.ShapeDtypeStruct((N, 1), jnp.float32),
    ]
    return pl.pallas_call(
        _tc_body, grid=grid, in_specs=in_specs, out_specs=out_specs,
        out_shape=out_shape)(sums, cnts, xp, xg, xs, *weights)


def kernel(x_pfas_sites, x_gw_wells, x_sw_stations,
           edge_index_pg, edge_index_gp, edge_index_ps, edge_index_sp,
           Wl_pg, bl_pg, Wr_pg, Wl_gp, bl_gp, Wr_gp,
           Wl_ps, bl_ps, Wr_ps, Wl_sp, bl_sp, Wr_sp,
           W_lin, b_lin, prelu_w):
    zrows = jnp.zeros((ROWS_PT, D), jnp.float32)
    zcnt = jnp.zeros((ROWS_PT,), jnp.float32)
    onesh = jnp.ones((CH,), jnp.float32)
    sums, cnts = _sc_aggregate(
        x_pfas_sites, x_gw_wells, x_sw_stations,
        edge_index_pg, edge_index_gp, edge_index_sp, edge_index_ps,
        zrows, zcnt, onesh)
    cnts4 = cnts.reshape(4, NC, NP, 1)
    out_pfas, gw, sw = _tc_dense(
        sums, cnts4, x_pfas_sites, x_gw_wells, x_sw_stations,
        Wl_pg, bl_pg, Wr_pg, Wl_gp, bl_gp, Wr_gp,
        Wl_ps, bl_ps, Wr_ps, Wl_sp, bl_sp, Wr_sp,
        W_lin, b_lin, prelu_w)
    return out_pfas, gw, sw
